# Initial kernel scaffold; baseline (speedup 1.0000x reference)
import jax
import jax.numpy as jnp
from jax.experimental import pallas as pl


def kernel(x, adj, W_rel, b_rel, W_root):
    raise NotImplementedError("write your pallas kernel here")


# speedup vs baseline: 2.8711x; 2.8711x over previous
import functools

import jax
import jax.numpy as jnp
from jax import lax
from jax.experimental import pallas as pl
from jax.experimental.pallas import tpu as pltpu
from jax.experimental.pallas import tpu_sc as plsc

N = 10000
D = 128
E = 320000

NC = 2
NS = 16
NW = NC * NS

CH = 128
E_PAD = 323584
EP = E_PAD // NW
NCHUNK = EP // CH

N_PAD = 10240
RPS = N_PAD // NS

def _i0():
    return jnp.int32(0)


BM = 1000
BD_I = 1024
BD_J = 1024


def _sc_scatter_body(h_hbm, src_hbm, dst_hbm, zero_hbm, out0, out1,
                     src_v, dst_v, rows_v, acc_sh, sem):
    cid = lax.axis_index("c")
    sid = lax.axis_index("s")
    gid = cid * jnp.int32(NS) + sid

    zsl = pl.ds(sid * RPS, RPS)
    pltpu.sync_copy(zero_hbm.at[zsl], acc_sh.at[zsl])
    plsc.subcore_barrier()

    base = gid * jnp.int32(EP)

    def chunk(c, carry):
        off = base + c * jnp.int32(CH)
        pltpu.sync_copy(src_hbm.at[pl.ds(off, CH)], src_v)
        pltpu.sync_copy(dst_hbm.at[pl.ds(off, CH)], dst_v)
        pltpu.async_copy(h_hbm.at[src_v], rows_v, sem).wait()
        pltpu.sync_copy(rows_v, acc_sh.at[dst_v], add=True)
        return carry

    lax.fori_loop(jnp.int32(0), jnp.int32(NCHUNK), chunk, jnp.int32(0))
    plsc.subcore_barrier()

    osl = pl.ds(sid * RPS, RPS)

    @pl.when(cid == 0)
    def _():
        pltpu.sync_copy(acc_sh.at[osl], out0.at[osl])

    @pl.when(cid == 1)
    def _():
        pltpu.sync_copy(acc_sh.at[osl], out1.at[osl])


def _sc_scatter(h, src, dst, zeros):
    mesh = plsc.VectorSubcoreMesh(core_axis_name="c", subcore_axis_name="s")
    f = functools.partial(
        pl.kernel,
        out_type=(
            jax.ShapeDtypeStruct((N_PAD, D), jnp.float32),
            jax.ShapeDtypeStruct((N_PAD, D), jnp.float32),
        ),
        mesh=mesh,
        scratch_types=[
            pltpu.VMEM((CH,), jnp.int32),
            pltpu.VMEM((CH,), jnp.int32),
            pltpu.VMEM((CH, D), jnp.float32),
            pltpu.VMEM_SHARED((N_PAD, D), jnp.float32),
            pltpu.SemaphoreType.DMA,
        ],
    )(_sc_scatter_body)
    return f(h, src, dst, zeros)


def _conv_body(p0_ref, p1_ref, x_ref, wr_ref, wt_ref, b_ref, o_ref, *, relu):
    agg = p0_ref[...] + p1_ref[...]
    y = (
        jnp.dot(agg, wr_ref[...], preferred_element_type=jnp.float32,
                precision=lax.Precision.HIGHEST)
        + b_ref[...]
        + jnp.dot(x_ref[...], wt_ref[...], preferred_element_type=jnp.float32,
                  precision=lax.Precision.HIGHEST)
    )
    if relu:
        y = jnp.maximum(y, 0.0)
    o_ref[...] = y


def _conv_tc(p0, p1, x, w_rel_t, w_root_t, b2d, relu):
    grid = (N // BM,)
    return pl.pallas_call(
        functools.partial(_conv_body, relu=relu),
        grid=grid,
        in_specs=[
            pl.BlockSpec((BM, D), lambda i: (i, _i0())),
            pl.BlockSpec((BM, D), lambda i: (i, _i0())),
            pl.BlockSpec((BM, D), lambda i: (i, _i0())),
            pl.BlockSpec((D, D), lambda i: (_i0(), _i0())),
            pl.BlockSpec((D, D), lambda i: (_i0(), _i0())),
            pl.BlockSpec((1, D), lambda i: (_i0(), _i0())),
        ],
        out_specs=pl.BlockSpec((BM, D), lambda i: (i, _i0())),
        out_shape=jax.ShapeDtypeStruct((N, D), jnp.float32),
    )(p0, p1, x, w_rel_t, w_root_t, b2d)


def _decoder_body(zi_ref, zj_ref, o_ref):
    logits = lax.dot_general(
        zi_ref[...], zj_ref[...],
        (((1,), (1,)), ((), ())),
        preferred_element_type=jnp.float32,
        precision=lax.Precision.HIGHEST,
    )
    o_ref[...] = 1.0 / (1.0 + jnp.exp(-logits))


def _decoder_tc(z):
    grid = (pl.cdiv(N, BD_I), pl.cdiv(N, BD_J))
    return pl.pallas_call(
        _decoder_body,
        grid=grid,
        in_specs=[
            pl.BlockSpec((BD_I, D), lambda i, j: (i, _i0())),
            pl.BlockSpec((BD_J, D), lambda i, j: (j, _i0())),
        ],
        out_specs=pl.BlockSpec((BD_I, BD_J), lambda i, j: (i, j)),
        out_shape=jax.ShapeDtypeStruct((N, N), jnp.float32),
        compiler_params=pltpu.CompilerParams(
            dimension_semantics=("parallel", "parallel"),
        ),
    )(z, z)


def kernel(x, adj, W_rel, b_rel, W_root):
    x = x.astype(jnp.float32)
    src = adj[0].astype(jnp.int32)
    dst = adj[1].astype(jnp.int32)
    pad = E_PAD - E
    src = jnp.concatenate([src, jnp.zeros((pad,), jnp.int32)])
    dst = jnp.concatenate([dst, jnp.full((pad,), N_PAD - 1, jnp.int32)])

    zeros = jnp.zeros((N_PAD, D), jnp.float32)
    w_rel_t = W_rel.astype(jnp.float32).T
    w_root_t = W_root.astype(jnp.float32).T
    b2d = b_rel.astype(jnp.float32).reshape(1, D)

    p0, p1 = _sc_scatter(x, src, dst, zeros)
    h1 = _conv_tc(p0, p1, x, w_rel_t, w_root_t, b2d, relu=True)
    q0, q1 = _sc_scatter(h1, src, dst, zeros)
    x2 = _conv_tc(q0, q1, h1, w_rel_t, w_root_t, b2d, relu=False)
    z_pad = jnp.pad(x2, ((0, N_PAD - N), (0, 0)))
    A = _decoder_tc(z_pad)
    return (A, x2)
